# baseline (speedup 1.0000x reference)
import jax
import jax.numpy as jnp
from jax import lax
from jax.experimental import pallas as pl
from jax.experimental.pallas import tpu as pltpu, tpu_sc as plsc

K = 1_000_000
D = 64
NW = 32
CH = 512
NFULL = K // CH
TAIL_BASE = NFULL * CH
TBASE2 = K - 128
ITERS = (NFULL + NW - 1) // NW
GSZ = 128
EPS = 1e-8
NEG = -1e30

_mesh = plsc.VectorSubcoreMesh(
    core_axis_name="c", subcore_axis_name="s", num_cores=2, num_subcores=16
)
_params = pltpu.CompilerParams(needs_layout_passes=False)


def _rsqrt(x):
    bits = plsc.bitcast(x, jnp.int32)
    y = plsc.bitcast(jnp.int32(0x5F3759DF) - (bits >> 1), jnp.float32)
    for _ in range(3):
        y = y * (1.5 - 0.5 * x * y * y)
    return y


def _prep_query(qbuf, qb, iota):
    qsq = jnp.zeros((16,), jnp.float32)
    for c in range(4):
        qc = qbuf[pl.ds(16 * c, 16)]
        qsq = qsq + qc * qc
        for l in range(16):
            qb[16 * c + l, :] = jnp.take(qc, (iota & 0) + l)
    qsq = qsq + jnp.take(qsq, iota ^ 8)
    qsq = qsq + jnp.take(qsq, iota ^ 4)
    qsq = qsq + jnp.take(qsq, iota ^ 2)
    qsq = qsq + jnp.take(qsq, iota ^ 1)
    return qsq


def _sorted_merge(rv, ri, sv, si):
    bv = lax.rev(sv, (0,))
    bi = lax.rev(si, (0,))
    keep = rv >= bv
    mv = jnp.where(keep, rv, bv)
    mi = jnp.where(keep, ri, bi)
    return tuple(plsc.sort_key_val(mv, mi))


def _score_body(q_hbm, et_hbm, tw_hbm, outv_hbm, outi_hbm,
                qbuf, qb, rbufa, rbufb, wbufa, wbufb, ovb, oib, sema, semb):
    wid = lax.axis_index("s") * 2 + lax.axis_index("c")
    iota = lax.iota(jnp.int32, 16)
    zero = jnp.zeros((16,), jnp.float32)

    pltpu.sync_copy(q_hbm, qbuf)
    qsq = _prep_query(qbuf, qb, iota)

    def dma_start(chunk, rb, wb, sem):
        base = pl.multiple_of(chunk * CH, CH)

        @pl.when(chunk < NFULL)
        def _():
            pltpu.async_copy(et_hbm.at[:, pl.ds(base, CH)], rb, sem)
            pltpu.async_copy(tw_hbm.at[pl.ds(base, CH)], wb, sem)

    def dma_wait(chunk, rb, wb, sem):
        base = pl.multiple_of(chunk * CH, CH)

        @pl.when(chunk < NFULL)
        def _():
            pltpu.make_async_copy(
                et_hbm.at[:, pl.ds(base, CH)], rb, sem).wait()
            pltpu.make_async_copy(
                tw_hbm.at[pl.ds(base, CH)], wb, sem).wait()

    def compute(chunk, rbuf, wbuf, carry):
        active = chunk < NFULL
        base = pl.multiple_of(chunk * CH, CH)
        rv, ri = carry
        for g in range(CH // GSZ):
            ge = g * GSZ

            def jbody(j, accs, ge=ge, rbuf=rbuf):
                qjv = qb[j, pl.ds(0, 16)]
                out = []
                for t in range(GSZ // 16):
                    col = rbuf[j, pl.ds(ge + t * 16, 16)]
                    out.append(accs[2 * t] + col * qjv)
                    out.append(accs[2 * t + 1] + col * col)
                return tuple(out)

            accs = lax.fori_loop(0, D, jbody, (zero,) * (GSZ // 8),
                                 unroll=3)
            svecs, gvecs = [], []
            smax = None
            for t in range(GSZ // 16):
                d, n = accs[2 * t], accs[2 * t + 1]
                off = ge + t * 16
                w = wbuf[pl.ds(off, 16)]
                gidx = base + off + iota
                y = jnp.minimum(_rsqrt(n * qsq), 1.0 / EPS)
                s = jnp.where(active, d * w * y, NEG)
                svecs.append(s)
                gvecs.append(gidx)
                smax = s if smax is None else jnp.maximum(smax, s)

            def do_merge(carry, svecs=svecs, gvecs=gvecs):
                rv, ri = carry
                thr = jnp.take(rv, iota & 0)

                def one(carry, s, gidx):
                    rv, ri = carry

                    def m(c, s=s, gidx=gidx):
                        rv, ri = c
                        sv, si = plsc.sort_key_val(s, gidx)
                        return _sorted_merge(rv, ri, sv, si)

                    return lax.cond(jnp.any(s > thr), m, lambda x: x,
                                    (rv, ri))

                for s, gidx in zip(svecs, gvecs):
                    carry = one(carry, s, gidx)
                return carry

            thr = jnp.take(rv, iota & 0)
            rv, ri = lax.cond(jnp.any(smax > thr), do_merge,
                              lambda x: x, (rv, ri))
        return rv, ri

    dma_start(wid, rbufa, wbufa, sema)

    def body2(cc, carry):
        ch0 = (2 * cc) * NW + wid
        ch1 = ch0 + NW
        ch2 = ch0 + 2 * NW
        dma_start(ch1, rbufb, wbufb, semb)
        dma_wait(ch0, rbufa, wbufa, sema)
        carry = compute(ch0, rbufa, wbufa, carry)
        dma_start(ch2, rbufa, wbufa, sema)
        dma_wait(ch1, rbufb, wbufb, semb)
        carry = compute(ch1, rbufb, wbufb, carry)
        return carry

    rv0 = jnp.full((16,), NEG, jnp.float32)
    ri0 = jnp.zeros((16,), jnp.int32)
    rv, ri = lax.fori_loop(0, (ITERS + 1) // 2, body2, (rv0, ri0))

    ovb[...] = rv
    oib[...] = ri
    pltpu.sync_copy(ovb, outv_hbm.at[wid])
    pltpu.sync_copy(oib, outi_hbm.at[wid])


_score = pl.kernel(
    _score_body,
    out_type=(
        jax.ShapeDtypeStruct((NW, 16), jnp.float32),
        jax.ShapeDtypeStruct((NW, 16), jnp.int32),
    ),
    mesh=_mesh,
    scratch_types=[
        pltpu.VMEM((D,), jnp.float32),
        pltpu.VMEM((D, 16), jnp.float32),
        pltpu.VMEM((D, CH), jnp.float32),
        pltpu.VMEM((D, CH), jnp.float32),
        pltpu.VMEM((CH,), jnp.float32),
        pltpu.VMEM((CH,), jnp.float32),
        pltpu.VMEM((16,), jnp.float32),
        pltpu.VMEM((16,), jnp.int32),
        pltpu.SemaphoreType.DMA,
        pltpu.SemaphoreType.DMA,
    ],
    compiler_params=_params,
)


def _merge_body(cv_hbm, ci_hbm, q_hbm, tt_hbm, twt_hbm, outv_hbm, outi_hbm,
                cvb, cib, qbuf, qb, tbuf, twb, ovb, oib):
    wid = lax.axis_index("s") * 2 + lax.axis_index("c")
    iota = lax.iota(jnp.int32, 16)
    zero = jnp.zeros((16,), jnp.float32)

    @pl.when(wid == 0)
    def _():
        pltpu.sync_copy(cv_hbm, cvb)
        pltpu.sync_copy(ci_hbm, cib)
        pltpu.sync_copy(q_hbm, qbuf)
        pltpu.sync_copy(tt_hbm, tbuf)
        pltpu.sync_copy(twt_hbm, twb)
        qsq = _prep_query(qbuf, qb, iota)

        rv = cvb[0, pl.ds(0, 16)]
        ri = cib[0, pl.ds(0, 16)]
        for j in range(1, NW):
            sv = cvb[j, pl.ds(0, 16)]
            si = cib[j, pl.ds(0, 16)]
            rv, ri = _sorted_merge(rv, ri, sv, si)

        def jbody(j, accs):
            qjv = qb[j, pl.ds(0, 16)]
            out = []
            for t in range(8):
                col = tbuf[j, pl.ds(t * 16, 16)]
                out.append(accs[2 * t] + col * qjv)
                out.append(accs[2 * t + 1] + col * col)
            return tuple(out)

        accs = lax.fori_loop(0, D, jbody, (zero,) * 16)
        for t in range(8):
            d, n = accs[2 * t], accs[2 * t + 1]
            w = twb[pl.ds(t * 16, 16)]
            gidx = TBASE2 + t * 16 + iota
            y = jnp.minimum(_rsqrt(n * qsq), 1.0 / EPS)
            s = jnp.where(gidx >= TAIL_BASE, d * w * y, NEG)
            sv, si = plsc.sort_key_val(s, gidx)
            rv, ri = _sorted_merge(rv, ri, sv, si)

        fv, fi = plsc.sort_key_val(rv, ri, descending=True)
        ovb[...] = fv
        oib[...] = fi
        pltpu.sync_copy(ovb, outv_hbm)
        pltpu.sync_copy(oib, outi_hbm)


_merge = pl.kernel(
    _merge_body,
    out_type=(
        jax.ShapeDtypeStruct((16,), jnp.float32),
        jax.ShapeDtypeStruct((16,), jnp.int32),
    ),
    mesh=_mesh,
    scratch_types=[
        pltpu.VMEM((NW, 16), jnp.float32),
        pltpu.VMEM((NW, 16), jnp.int32),
        pltpu.VMEM((D,), jnp.float32),
        pltpu.VMEM((D, 16), jnp.float32),
        pltpu.VMEM((D, 128), jnp.float32),
        pltpu.VMEM((128,), jnp.float32),
        pltpu.VMEM((16,), jnp.float32),
        pltpu.VMEM((16,), jnp.int32),
    ],
    compiler_params=_params,
)


def kernel(query_embedding, episode_embeddings, temporal_weights, top_k):
    del top_k
    et = episode_embeddings.T
    cv, ci = _score(query_embedding, et, temporal_weights)
    tail_t = lax.slice(episode_embeddings, (TBASE2, 0), (K, D)).T
    tw_tail = lax.slice(temporal_weights, (TBASE2,), (K,))
    fv, fi = _merge(cv, ci, query_embedding, tail_t, tw_tail)
    return fv[:5], fi[:5]


# speedup vs baseline: 1.0874x; 1.0874x over previous
import jax
import jax.numpy as jnp
from jax import lax
from jax.experimental import pallas as pl
from jax.experimental.pallas import tpu as pltpu, tpu_sc as plsc

K = 1_000_000
D = 64
NW = 32
CH = 512
NFULL = K // CH
TAIL_BASE = NFULL * CH
TBASE2 = K - 128
ITERS = (NFULL + NW - 1) // NW
GSZ = 128
EPS = 1e-8
NEG = -1e30

_mesh = plsc.VectorSubcoreMesh(
    core_axis_name="c", subcore_axis_name="s", num_cores=2, num_subcores=16
)
_params = pltpu.CompilerParams(needs_layout_passes=False)


def _rsqrt(x):
    bits = plsc.bitcast(x, jnp.int32)
    y = plsc.bitcast(jnp.int32(0x5F3759DF) - (bits >> 1), jnp.float32)
    for _ in range(3):
        y = y * (1.5 - 0.5 * x * y * y)
    return y


def _prep_query(qbuf, qb, iota):
    qsq = jnp.zeros((16,), jnp.float32)
    for c in range(4):
        qc = qbuf[pl.ds(16 * c, 16)]
        qsq = qsq + qc * qc
        for l in range(16):
            qb[16 * c + l, :] = jnp.take(qc, (iota & 0) + l)
    qsq = qsq + jnp.take(qsq, iota ^ 8)
    qsq = qsq + jnp.take(qsq, iota ^ 4)
    qsq = qsq + jnp.take(qsq, iota ^ 2)
    qsq = qsq + jnp.take(qsq, iota ^ 1)
    return qsq


def _sorted_merge(rv, ri, sv, si):
    bv = lax.rev(sv, (0,))
    bi = lax.rev(si, (0,))
    keep = rv >= bv
    mv = jnp.where(keep, rv, bv)
    mi = jnp.where(keep, ri, bi)
    return tuple(plsc.sort_key_val(mv, mi))


def _score_body(q_hbm, et_hbm, tw_hbm, outv_hbm, outi_hbm,
                qbuf, qb, rbufa, rbufb, wbufa, wbufb, ovb, oib, sema, semb):
    wid = lax.axis_index("s") * 2 + lax.axis_index("c")
    iota = lax.iota(jnp.int32, 16)
    zero = jnp.zeros((16,), jnp.float32)

    pltpu.sync_copy(q_hbm, qbuf)
    qsq = _prep_query(qbuf, qb, iota)

    def dma_start(chunk, rb, wb, sem):
        base = pl.multiple_of(chunk * CH, CH)

        @pl.when(chunk < NFULL)
        def _():
            pltpu.async_copy(et_hbm.at[:, pl.ds(base, CH)], rb, sem)
            pltpu.async_copy(tw_hbm.at[pl.ds(base, CH)], wb, sem)

    def dma_wait(chunk, rb, wb, sem):
        base = pl.multiple_of(chunk * CH, CH)

        @pl.when(chunk < NFULL)
        def _():
            pltpu.make_async_copy(
                et_hbm.at[:, pl.ds(base, CH)], rb, sem).wait()
            pltpu.make_async_copy(
                tw_hbm.at[pl.ds(base, CH)], wb, sem).wait()

    def compute(chunk, rbuf, wbuf, carry):
        active = chunk < NFULL
        base = pl.multiple_of(chunk * CH, CH)
        rv, ri = carry
        for g in range(CH // GSZ):
            ge = g * GSZ

            def jbody(j, accs, ge=ge, rbuf=rbuf):
                qjv = qb[j, pl.ds(0, 16)]
                out = []
                for t in range(GSZ // 16):
                    col = rbuf[j, pl.ds(ge + t * 16, 16)]
                    out.append(accs[2 * t] + col * qjv)
                    out.append(accs[2 * t + 1] + col * col)
                return tuple(out)

            accs = lax.fori_loop(0, D, jbody, (zero,) * (GSZ // 8),
                                 unroll=2)
            svecs, gvecs = [], []
            smax = None
            for t in range(GSZ // 16):
                d, n = accs[2 * t], accs[2 * t + 1]
                off = ge + t * 16
                w = wbuf[pl.ds(off, 16)]
                gidx = base + off + iota
                y = jnp.minimum(_rsqrt(n * qsq), 1.0 / EPS)
                s = jnp.where(active, d * w * y, NEG)
                svecs.append(s)
                gvecs.append(gidx)
                smax = s if smax is None else jnp.maximum(smax, s)

            def do_merge(carry, svecs=svecs, gvecs=gvecs):
                rv, ri = carry
                thr = jnp.take(rv, iota & 0)

                def one(carry, s, gidx):
                    rv, ri = carry

                    def m(c, s=s, gidx=gidx):
                        rv, ri = c
                        sv, si = plsc.sort_key_val(s, gidx)
                        return _sorted_merge(rv, ri, sv, si)

                    return lax.cond(jnp.any(s > thr), m, lambda x: x,
                                    (rv, ri))

                for s, gidx in zip(svecs, gvecs):
                    carry = one(carry, s, gidx)
                return carry

            thr = jnp.take(rv, iota & 0)
            rv, ri = lax.cond(jnp.any(smax > thr), do_merge,
                              lambda x: x, (rv, ri))
        return rv, ri

    dma_start(wid, rbufa, wbufa, sema)

    def body2(cc, carry):
        ch0 = (2 * cc) * NW + wid
        ch1 = ch0 + NW
        ch2 = ch0 + 2 * NW
        dma_start(ch1, rbufb, wbufb, semb)
        dma_wait(ch0, rbufa, wbufa, sema)
        carry = compute(ch0, rbufa, wbufa, carry)
        dma_start(ch2, rbufa, wbufa, sema)
        dma_wait(ch1, rbufb, wbufb, semb)
        carry = compute(ch1, rbufb, wbufb, carry)
        return carry

    rv0 = jnp.full((16,), NEG, jnp.float32)
    ri0 = jnp.zeros((16,), jnp.int32)
    rv, ri = lax.fori_loop(0, (ITERS + 1) // 2, body2, (rv0, ri0))

    ovb[...] = rv
    oib[...] = ri
    pltpu.sync_copy(ovb, outv_hbm.at[wid])
    pltpu.sync_copy(oib, outi_hbm.at[wid])


_score = pl.kernel(
    _score_body,
    out_type=(
        jax.ShapeDtypeStruct((NW, 16), jnp.float32),
        jax.ShapeDtypeStruct((NW, 16), jnp.int32),
    ),
    mesh=_mesh,
    scratch_types=[
        pltpu.VMEM((D,), jnp.float32),
        pltpu.VMEM((D, 16), jnp.float32),
        pltpu.VMEM((D, CH), jnp.float32),
        pltpu.VMEM((D, CH), jnp.float32),
        pltpu.VMEM((CH,), jnp.float32),
        pltpu.VMEM((CH,), jnp.float32),
        pltpu.VMEM((16,), jnp.float32),
        pltpu.VMEM((16,), jnp.int32),
        pltpu.SemaphoreType.DMA,
        pltpu.SemaphoreType.DMA,
    ],
    compiler_params=_params,
)


def _merge_body(cv_hbm, ci_hbm, q_hbm, tt_hbm, twt_hbm, outv_hbm, outi_hbm,
                cvb, cib, qbuf, qb, tbuf, twb, ovb, oib):
    wid = lax.axis_index("s") * 2 + lax.axis_index("c")
    iota = lax.iota(jnp.int32, 16)
    zero = jnp.zeros((16,), jnp.float32)

    @pl.when(wid == 0)
    def _():
        pltpu.sync_copy(cv_hbm, cvb)
        pltpu.sync_copy(ci_hbm, cib)
        pltpu.sync_copy(q_hbm, qbuf)
        pltpu.sync_copy(tt_hbm, tbuf)
        pltpu.sync_copy(twt_hbm, twb)
        qsq = _prep_query(qbuf, qb, iota)

        rv = cvb[0, pl.ds(0, 16)]
        ri = cib[0, pl.ds(0, 16)]
        for j in range(1, NW):
            sv = cvb[j, pl.ds(0, 16)]
            si = cib[j, pl.ds(0, 16)]
            rv, ri = _sorted_merge(rv, ri, sv, si)

        def jbody(j, accs):
            qjv = qb[j, pl.ds(0, 16)]
            out = []
            for t in range(8):
                col = tbuf[j, pl.ds(t * 16, 16)]
                out.append(accs[2 * t] + col * qjv)
                out.append(accs[2 * t + 1] + col * col)
            return tuple(out)

        accs = lax.fori_loop(0, D, jbody, (zero,) * 16)
        for t in range(8):
            d, n = accs[2 * t], accs[2 * t + 1]
            w = twb[pl.ds(t * 16, 16)]
            gidx = TBASE2 + t * 16 + iota
            y = jnp.minimum(_rsqrt(n * qsq), 1.0 / EPS)
            s = jnp.where(gidx >= TAIL_BASE, d * w * y, NEG)
            sv, si = plsc.sort_key_val(s, gidx)
            rv, ri = _sorted_merge(rv, ri, sv, si)

        fv, fi = plsc.sort_key_val(rv, ri, descending=True)
        ovb[...] = fv
        oib[...] = fi
        pltpu.sync_copy(ovb, outv_hbm)
        pltpu.sync_copy(oib, outi_hbm)


_merge = pl.kernel(
    _merge_body,
    out_type=(
        jax.ShapeDtypeStruct((16,), jnp.float32),
        jax.ShapeDtypeStruct((16,), jnp.int32),
    ),
    mesh=_mesh,
    scratch_types=[
        pltpu.VMEM((NW, 16), jnp.float32),
        pltpu.VMEM((NW, 16), jnp.int32),
        pltpu.VMEM((D,), jnp.float32),
        pltpu.VMEM((D, 16), jnp.float32),
        pltpu.VMEM((D, 128), jnp.float32),
        pltpu.VMEM((128,), jnp.float32),
        pltpu.VMEM((16,), jnp.float32),
        pltpu.VMEM((16,), jnp.int32),
    ],
    compiler_params=_params,
)


def kernel(query_embedding, episode_embeddings, temporal_weights, top_k):
    del top_k
    et = episode_embeddings.T
    cv, ci = _score(query_embedding, et, temporal_weights)
    tail_t = lax.slice(episode_embeddings, (TBASE2, 0), (K, D)).T
    tw_tail = lax.slice(temporal_weights, (TBASE2,), (K,))
    fv, fi = _merge(cv, ci, query_embedding, tail_t, tw_tail)
    return fv[:5], fi[:5]
